# baseline (speedup 1.0000x reference)
import jax
import jax.numpy as jnp
from jax import lax
from jax.experimental import pallas as pl
from jax.experimental.pallas import tpu as pltpu
from jax.experimental.pallas import tpu_sc as plsc

WORD_DIM = 64
SMALL_DIM = 16
FILL_DIM = 3 * SMALL_DIM
OUT_DIM = WORD_DIM + FILL_DIM

NUM_WORKERS = 32
GRP = 128
LANES = 16


def _sc_body(word_idx, bio_idx, f0_idx, f1_idx, w_word, wb_t, wf_t, out,
             idx_w, idx_b, idx_0, idx_1, wb_v, wf_v,
             wbuf_a, wbuf_b, fbuf_a, fbuf_b,
             sem_i, gsem_a, gsem_b, wsem_a, wsem_b):
    groups = word_idx.shape[0] // NUM_WORKERS
    pairs = groups // 2
    per_worker = groups * GRP

    wid = lax.axis_index("s") * 2 + lax.axis_index("c")
    worker_base = wid * per_worker
    wrows = pl.ds(pl.multiple_of(wid * groups, groups), groups)
    wflat = pl.ds(pl.multiple_of(wid * per_worker, per_worker), per_worker)

    loads = [
        pltpu.async_copy(word_idx.at[wrows, :], idx_w, sem_i),
        pltpu.async_copy(bio_idx.at[wflat], idx_b, sem_i),
        pltpu.async_copy(f0_idx.at[wflat], idx_0, sem_i),
        pltpu.async_copy(f1_idx.at[wflat], idx_1, sem_i),
        pltpu.async_copy(wb_t, wb_v, sem_i),
        pltpu.async_copy(wf_t, wf_v, sem_i),
    ]
    for cp in loads:
        cp.wait()

    def fire_gather(g, buf, sem):
        return pltpu.async_copy(w_word.at[idx_w.at[g]], buf, sem)

    def drain_gather(g, buf, sem):
        pltpu.make_async_copy(w_word.at[idx_w.at[g]], buf, sem).wait()

    def fire_write(g, fbuf, sem):
        return pltpu.async_copy(
            fbuf, out.at[pl.ds(worker_base + g * GRP, GRP)], sem)

    def drain_write(g, fbuf, sem):
        pltpu.make_async_copy(
            fbuf, out.at[pl.ds(worker_base + g * GRP, GRP)], sem).wait()

    def copy_word(wbuf, fbuf):
        def row(r, carry):
            for k in range(WORD_DIM // LANES):
                fbuf[r, pl.ds(k * LANES, LANES)] = wbuf[r, pl.ds(k * LANES, LANES)]
            return carry
        lax.fori_loop(0, GRP, row, 0)

    iota = lax.iota(jnp.int32, LANES)

    def fill(g, fbuf):
        def vgroup(v, carry):
            o = g * GRP + v * LANES
            rows = v * LANES + iota
            lanes_b = idx_b[pl.ds(o, LANES)]
            lanes_0 = idx_0[pl.ds(o, LANES)]
            lanes_1 = idx_1[pl.ds(o, LANES)]
            for c in range(SMALL_DIM):
                cvec = jnp.full((LANES,), c, jnp.int32)
                vb = plsc.load_gather(wb_v, [cvec, lanes_b])
                plsc.store_scatter(
                    fbuf, [rows, jnp.full((LANES,), WORD_DIM + c, jnp.int32)], vb)
                v0 = plsc.load_gather(wf_v, [cvec, lanes_0])
                plsc.store_scatter(
                    fbuf, [rows, jnp.full((LANES,), WORD_DIM + SMALL_DIM + c, jnp.int32)], v0)
                v1 = plsc.load_gather(wf_v, [cvec, lanes_1])
                plsc.store_scatter(
                    fbuf, [rows, jnp.full((LANES,), WORD_DIM + 2 * SMALL_DIM + c, jnp.int32)], v1)
            return carry

        lax.fori_loop(0, GRP // LANES, vgroup, 0)

    fire_gather(0, wbuf_a, gsem_a)

    def pair_body(p, carry):
        a = 2 * p
        b = a + 1
        @pl.when(p > 0)
        def _():
            drain_write(a - 1, fbuf_b, wsem_b)

        fire_gather(b, wbuf_b, gsem_b)
        fill(a, fbuf_a)
        drain_gather(a, wbuf_a, gsem_a)
        copy_word(wbuf_a, fbuf_a)
        fire_write(a, fbuf_a, wsem_a)
        fire_gather(jnp.minimum(b + 1, groups - 1), wbuf_a, gsem_a)
        fill(b, fbuf_b)
        drain_gather(b, wbuf_b, gsem_b)
        copy_word(wbuf_b, fbuf_b)
        drain_write(a, fbuf_a, wsem_a)
        fire_write(b, fbuf_b, wsem_b)
        return carry

    lax.fori_loop(0, pairs, pair_body, 0)
    drain_gather(groups - 1, wbuf_a, gsem_a)
    drain_write(groups - 1, fbuf_b, wsem_b)


def kernel(word_index, bio_index, feat_index_0, feat_index_1, W_word, W_bio, W_feat):
    B, S = word_index.shape
    n = B * S
    groups_total = n // GRP
    wf = word_index.reshape(groups_total, GRP).astype(jnp.int32)
    bf = bio_index.reshape(n).astype(jnp.int32)
    f0 = feat_index_0.reshape(n).astype(jnp.int32)
    f1 = feat_index_1.reshape(n).astype(jnp.int32)
    wb_t = W_bio.T
    wf_t = W_feat.T
    groups = groups_total // NUM_WORKERS
    per_worker = groups * GRP

    mesh = plsc.VectorSubcoreMesh(
        core_axis_name="c", subcore_axis_name="s", num_cores=2, num_subcores=16)

    run = pl.kernel(
        _sc_body,
        out_type=jax.ShapeDtypeStruct((n, 128), jnp.float32),
        mesh=mesh,
        scratch_types=[
            pltpu.VMEM((groups, GRP), jnp.int32),
            pltpu.VMEM((per_worker,), jnp.int32),
            pltpu.VMEM((per_worker,), jnp.int32),
            pltpu.VMEM((per_worker,), jnp.int32),
            pltpu.VMEM((LANES, 8), jnp.float32),
            pltpu.VMEM((LANES, 1000), jnp.float32),
            pltpu.VMEM((GRP, WORD_DIM), jnp.float32),
            pltpu.VMEM((GRP, WORD_DIM), jnp.float32),
            pltpu.VMEM((GRP, 128), jnp.float32),
            pltpu.VMEM((GRP, 128), jnp.float32),
            pltpu.SemaphoreType.DMA,
            pltpu.SemaphoreType.DMA,
            pltpu.SemaphoreType.DMA,
            pltpu.SemaphoreType.DMA,
            pltpu.SemaphoreType.DMA,
        ],
        compiler_params=pltpu.CompilerParams(
            use_tc_tiling_on_sc=False, needs_layout_passes=False),
    )
    out = run(wf, bf, f0, f1, W_word, wb_t, wf_t)
    return out[:, :OUT_DIM].reshape(B, S, OUT_DIM)


# speedup vs baseline: 1.0770x; 1.0770x over previous
import jax
import jax.numpy as jnp
from jax import lax
from jax.experimental import pallas as pl
from jax.experimental.pallas import tpu as pltpu
from jax.experimental.pallas import tpu_sc as plsc

WORD_DIM = 64
SMALL_DIM = 16
FILL_DIM = 3 * SMALL_DIM
OUT_DIM = WORD_DIM + FILL_DIM

NUM_WORKERS = 32
GRP = 128
LANES = 16


def _sc_body(word_idx, bio_idx, f0_idx, f1_idx, w_word, wb_t, wf_t, out,
             idx_w, idx_b, idx_0, idx_1, wb_v, wf_v,
             wbuf_a, wbuf_b, fbuf_a, fbuf_b,
             sem_i, gsem_a, gsem_b, wsem_a, wsem_b):
    groups = word_idx.shape[0] // NUM_WORKERS
    pairs = groups // 2
    per_worker = groups * GRP

    wid = lax.axis_index("s") * 2 + lax.axis_index("c")
    worker_base = wid * per_worker
    wrows = pl.ds(pl.multiple_of(wid * groups, groups), groups)
    wflat = pl.ds(pl.multiple_of(wid * per_worker, per_worker), per_worker)

    loads = [
        pltpu.async_copy(word_idx.at[wrows, :], idx_w, sem_i),
        pltpu.async_copy(bio_idx.at[wflat], idx_b, sem_i),
        pltpu.async_copy(f0_idx.at[wflat], idx_0, sem_i),
        pltpu.async_copy(f1_idx.at[wflat], idx_1, sem_i),
        pltpu.async_copy(wb_t, wb_v, sem_i),
        pltpu.async_copy(wf_t, wf_v, sem_i),
    ]
    for cp in loads:
        cp.wait()

    def fire_gather(g, buf, sem):
        return pltpu.async_copy(w_word.at[idx_w.at[g]], buf, sem)

    def drain_gather(g, buf, sem):
        pltpu.make_async_copy(w_word.at[idx_w.at[g]], buf, sem).wait()

    def out_slices(g):
        rows = pl.ds(worker_base + g * GRP, GRP)
        return (out.at[rows, pl.ds(0, WORD_DIM)],
                out.at[rows, pl.ds(WORD_DIM, WORD_DIM)])

    def fire_write(g, wbuf, fbuf, sem):
        dst_w, dst_f = out_slices(g)
        return [pltpu.async_copy(wbuf, dst_w, sem),
                pltpu.async_copy(fbuf.at[:, pl.ds(WORD_DIM, WORD_DIM)], dst_f, sem)]

    def drain_write(g, wbuf, fbuf, sem):
        dst_w, dst_f = out_slices(g)
        pltpu.make_async_copy(wbuf, dst_w, sem).wait()
        pltpu.make_async_copy(
            fbuf.at[:, pl.ds(WORD_DIM, WORD_DIM)], dst_f, sem).wait()

    iota = lax.iota(jnp.int32, LANES)

    def fill(g, fbuf):
        def vgroup(v, carry):
            o = g * GRP + v * LANES
            rows = v * LANES + iota
            lanes_b = idx_b[pl.ds(o, LANES)]
            lanes_0 = idx_0[pl.ds(o, LANES)]
            lanes_1 = idx_1[pl.ds(o, LANES)]
            for c in range(SMALL_DIM):
                cvec = jnp.full((LANES,), c, jnp.int32)
                vb = plsc.load_gather(wb_v, [cvec, lanes_b])
                plsc.store_scatter(
                    fbuf, [rows, jnp.full((LANES,), WORD_DIM + c, jnp.int32)], vb)
                v0 = plsc.load_gather(wf_v, [cvec, lanes_0])
                plsc.store_scatter(
                    fbuf, [rows, jnp.full((LANES,), WORD_DIM + SMALL_DIM + c, jnp.int32)], v0)
                v1 = plsc.load_gather(wf_v, [cvec, lanes_1])
                plsc.store_scatter(
                    fbuf, [rows, jnp.full((LANES,), WORD_DIM + 2 * SMALL_DIM + c, jnp.int32)], v1)
            return carry

        lax.fori_loop(0, GRP // LANES, vgroup, 0)

    fire_gather(0, wbuf_a, gsem_a)

    def pair_body(p, carry):
        a = 2 * p
        b = a + 1
        fill(a, fbuf_a)

        @pl.when(p > 0)
        def _():
            drain_write(a - 1, wbuf_b, fbuf_b, wsem_b)

        fire_gather(b, wbuf_b, gsem_b)
        drain_gather(a, wbuf_a, gsem_a)
        fire_write(a, wbuf_a, fbuf_a, wsem_a)
        fill(b, fbuf_b)
        drain_write(a, wbuf_a, fbuf_a, wsem_a)
        fire_gather(jnp.minimum(b + 1, groups - 1), wbuf_a, gsem_a)
        drain_gather(b, wbuf_b, gsem_b)
        fire_write(b, wbuf_b, fbuf_b, wsem_b)
        return carry

    lax.fori_loop(0, pairs, pair_body, 0)
    drain_gather(groups - 1, wbuf_a, gsem_a)
    drain_write(groups - 1, wbuf_b, fbuf_b, wsem_b)


def kernel(word_index, bio_index, feat_index_0, feat_index_1, W_word, W_bio, W_feat):
    B, S = word_index.shape
    n = B * S
    groups_total = n // GRP
    wf = word_index.reshape(groups_total, GRP).astype(jnp.int32)
    bf = bio_index.reshape(n).astype(jnp.int32)
    f0 = feat_index_0.reshape(n).astype(jnp.int32)
    f1 = feat_index_1.reshape(n).astype(jnp.int32)
    wb_t = W_bio.T
    wf_t = W_feat.T
    groups = groups_total // NUM_WORKERS
    per_worker = groups * GRP

    mesh = plsc.VectorSubcoreMesh(
        core_axis_name="c", subcore_axis_name="s", num_cores=2, num_subcores=16)

    run = pl.kernel(
        _sc_body,
        out_type=jax.ShapeDtypeStruct((n, 128), jnp.float32),
        mesh=mesh,
        scratch_types=[
            pltpu.VMEM((groups, GRP), jnp.int32),
            pltpu.VMEM((per_worker,), jnp.int32),
            pltpu.VMEM((per_worker,), jnp.int32),
            pltpu.VMEM((per_worker,), jnp.int32),
            pltpu.VMEM((LANES, 8), jnp.float32),
            pltpu.VMEM((LANES, 1000), jnp.float32),
            pltpu.VMEM((GRP, WORD_DIM), jnp.float32),
            pltpu.VMEM((GRP, WORD_DIM), jnp.float32),
            pltpu.VMEM((GRP, 128), jnp.float32),
            pltpu.VMEM((GRP, 128), jnp.float32),
            pltpu.SemaphoreType.DMA,
            pltpu.SemaphoreType.DMA,
            pltpu.SemaphoreType.DMA,
            pltpu.SemaphoreType.DMA,
            pltpu.SemaphoreType.DMA,
        ],
        compiler_params=pltpu.CompilerParams(
            use_tc_tiling_on_sc=False, needs_layout_passes=False),
    )
    out = run(wf, bf, f0, f1, W_word, wb_t, wf_t)
    return out[:, :OUT_DIM].reshape(B, S, OUT_DIM)


# speedup vs baseline: 1.1245x; 1.0441x over previous
import jax
import jax.numpy as jnp
from jax import lax
from jax.experimental import pallas as pl
from jax.experimental.pallas import tpu as pltpu
from jax.experimental.pallas import tpu_sc as plsc

WORD_DIM = 64
SMALL_DIM = 16
FILL_DIM = 3 * SMALL_DIM
OUT_DIM = WORD_DIM + FILL_DIM

NUM_WORKERS = 32
GRP = 128
LANES = 16


def _sc_body(word_idx, bio_idx, f0_idx, f1_idx, w_word, wb_t, wf_t, out,
             idx_w, idx_b, idx_0, idx_1, wb_v, wf_v,
             wbuf_a, wbuf_b, fbuf_a, fbuf_b,
             sem_i, gsem_a, gsem_b, wsem_a, wsem_b):
    groups = word_idx.shape[0] // NUM_WORKERS
    pairs = groups // 2
    per_worker = groups * GRP

    wid = lax.axis_index("s") * 2 + lax.axis_index("c")
    worker_base = wid * per_worker
    wrows = pl.ds(pl.multiple_of(wid * groups, groups), groups)
    wflat = pl.ds(pl.multiple_of(wid * per_worker, per_worker), per_worker)

    loads = [
        pltpu.async_copy(word_idx.at[wrows, :], idx_w, sem_i),
        pltpu.async_copy(bio_idx.at[wflat], idx_b, sem_i),
        pltpu.async_copy(f0_idx.at[wflat], idx_0, sem_i),
        pltpu.async_copy(f1_idx.at[wflat], idx_1, sem_i),
        pltpu.async_copy(wb_t, wb_v, sem_i),
        pltpu.async_copy(wf_t, wf_v, sem_i),
    ]
    for cp in loads:
        cp.wait()

    def fire_gather(g, buf, sem):
        return pltpu.async_copy(w_word.at[idx_w.at[g]], buf, sem)

    def drain_gather(g, buf, sem):
        pltpu.make_async_copy(w_word.at[idx_w.at[g]], buf, sem).wait()

    def out_slices(g):
        rows = pl.ds(worker_base + g * GRP, GRP)
        return (out.at[rows, pl.ds(0, WORD_DIM)],
                out.at[rows, pl.ds(WORD_DIM, WORD_DIM)])

    def fire_write(g, wbuf, fbuf, sem):
        dst_w, dst_f = out_slices(g)
        return [pltpu.async_copy(wbuf, dst_w, sem),
                pltpu.async_copy(fbuf, dst_f, sem)]

    def drain_write(g, wbuf, fbuf, sem):
        dst_w, dst_f = out_slices(g)
        pltpu.make_async_copy(wbuf, dst_w, sem).wait()
        pltpu.make_async_copy(fbuf, dst_f, sem).wait()

    iota = lax.iota(jnp.int32, LANES)

    def fill(g, fbuf):
        def vgroup(v, carry):
            o = g * GRP + v * LANES
            rows = v * LANES + iota
            lanes_b = idx_b[pl.ds(o, LANES)]
            lanes_0 = idx_0[pl.ds(o, LANES)]
            lanes_1 = idx_1[pl.ds(o, LANES)]
            for c in range(SMALL_DIM):
                cvec = jnp.full((LANES,), c, jnp.int32)
                vb = plsc.load_gather(wb_v, [cvec, lanes_b])
                plsc.store_scatter(
                    fbuf, [rows, jnp.full((LANES,), c, jnp.int32)], vb)
                v0 = plsc.load_gather(wf_v, [cvec, lanes_0])
                plsc.store_scatter(
                    fbuf, [rows, jnp.full((LANES,), SMALL_DIM + c, jnp.int32)], v0)
                v1 = plsc.load_gather(wf_v, [cvec, lanes_1])
                plsc.store_scatter(
                    fbuf, [rows, jnp.full((LANES,), 2 * SMALL_DIM + c, jnp.int32)], v1)
            return carry

        lax.fori_loop(0, GRP // LANES, vgroup, 0)

    fire_gather(0, wbuf_a, gsem_a)

    def pair_body(p, carry):
        a = 2 * p
        b = a + 1
        fill(a, fbuf_a)

        @pl.when(p > 0)
        def _():
            drain_write(a - 1, wbuf_b, fbuf_b, wsem_b)

        fire_gather(b, wbuf_b, gsem_b)
        drain_gather(a, wbuf_a, gsem_a)
        fire_write(a, wbuf_a, fbuf_a, wsem_a)
        fill(b, fbuf_b)
        drain_write(a, wbuf_a, fbuf_a, wsem_a)
        fire_gather(jnp.minimum(b + 1, groups - 1), wbuf_a, gsem_a)
        drain_gather(b, wbuf_b, gsem_b)
        fire_write(b, wbuf_b, fbuf_b, wsem_b)
        return carry

    lax.fori_loop(0, pairs, pair_body, 0)
    drain_gather(groups - 1, wbuf_a, gsem_a)
    drain_write(groups - 1, wbuf_b, fbuf_b, wsem_b)


def kernel(word_index, bio_index, feat_index_0, feat_index_1, W_word, W_bio, W_feat):
    B, S = word_index.shape
    n = B * S
    groups_total = n // GRP
    wf = word_index.reshape(groups_total, GRP).astype(jnp.int32)
    bf = bio_index.reshape(n).astype(jnp.int32)
    f0 = feat_index_0.reshape(n).astype(jnp.int32)
    f1 = feat_index_1.reshape(n).astype(jnp.int32)
    wb_t = W_bio.T
    wf_t = W_feat.T
    groups = groups_total // NUM_WORKERS
    per_worker = groups * GRP

    mesh = plsc.VectorSubcoreMesh(
        core_axis_name="c", subcore_axis_name="s", num_cores=2, num_subcores=16)

    run = pl.kernel(
        _sc_body,
        out_type=jax.ShapeDtypeStruct((n, 128), jnp.float32),
        mesh=mesh,
        scratch_types=[
            pltpu.VMEM((groups, GRP), jnp.int32),
            pltpu.VMEM((per_worker,), jnp.int32),
            pltpu.VMEM((per_worker,), jnp.int32),
            pltpu.VMEM((per_worker,), jnp.int32),
            pltpu.VMEM((LANES, 8), jnp.float32),
            pltpu.VMEM((LANES, 1000), jnp.float32),
            pltpu.VMEM((GRP, WORD_DIM), jnp.float32),
            pltpu.VMEM((GRP, WORD_DIM), jnp.float32),
            pltpu.VMEM((GRP, WORD_DIM), jnp.float32),
            pltpu.VMEM((GRP, WORD_DIM), jnp.float32),
            pltpu.SemaphoreType.DMA,
            pltpu.SemaphoreType.DMA,
            pltpu.SemaphoreType.DMA,
            pltpu.SemaphoreType.DMA,
            pltpu.SemaphoreType.DMA,
        ],
        compiler_params=pltpu.CompilerParams(
            use_tc_tiling_on_sc=False, needs_layout_passes=False),
    )
    out = run(wf, bf, f0, f1, W_word, wb_t, wf_t)
    return out[:, :OUT_DIM].reshape(B, S, OUT_DIM)
